# baseline (speedup 1.0000x reference)
import jax
import jax.numpy as jnp
from jax import lax
from jax.experimental import pallas as pl
from jax.experimental.pallas import tpu as pltpu
from jax.experimental.pallas import tpu_sc as plsc

_VOCAB = 100000
_DIM = 64
_BATCH = 1024
_CTX = 50

_NC = 2
_NS = 16
_NW = _NC * _NS
_LANES = 16

_D_PER_W = _DIM // _NW
_BCHUNK = 128
_NBCHUNK = _BATCH // _BCHUNK
_NGROUP = _BCHUNK // _LANES


def _sc_pool_kernel(
    ctxT_hbm, tableT_hbm, out_hbm, row_v, idx0_v, idx1_v, out_v, semr, sem0, sem1
):
    wid = lax.axis_index("s") * _NC + lax.axis_index("c")
    inv_ctx = jnp.float32(1.0 / _CTX)
    sems = (sem0, sem1)
    bufs = (idx0_v, idx1_v)

    def stage_idx(c):
        return pltpu.async_copy(
            ctxT_hbm.at[:, pl.ds(c * _BCHUNK, _BCHUNK)],
            bufs[c % 2],
            sems[c % 2],
        )

    for k in range(_D_PER_W):
        d = wid * _D_PER_W + k
        row_cp = pltpu.async_copy(tableT_hbm.at[d], row_v, semr)
        idx_cp = stage_idx(0)
        for c in range(_NBCHUNK):
            idx_cp.wait()
            if c + 1 < _NBCHUNK:
                idx_cp = stage_idx(c + 1)
            if c == 0:
                row_cp.wait()
            buf = c % 2

            def jbody(j, accs):
                return tuple(
                    accs[g]
                    + plsc.load_gather(
                        row_v, [bufs[buf][j, pl.ds(g * _LANES, _LANES)]]
                    )
                    for g in range(_NGROUP)
                )

            accs = tuple(
                jnp.zeros((_LANES,), jnp.float32) for _ in range(_NGROUP)
            )
            accs = lax.fori_loop(0, _CTX, jbody, accs, unroll=5)
            for g in range(_NGROUP):
                out_v[k, pl.ds(c * _BCHUNK + g * _LANES, _LANES)] = (
                    accs[g] * inv_ctx
                )

    pltpu.sync_copy(out_v, out_hbm.at[pl.ds(wid * _D_PER_W, _D_PER_W)])


@jax.jit
def _sc_pool_t(ctxT, tableT):
    mesh = plsc.VectorSubcoreMesh(
        core_axis_name="c", subcore_axis_name="s",
        num_cores=_NC, num_subcores=_NS,
    )
    return pl.kernel(
        _sc_pool_kernel,
        out_type=jax.ShapeDtypeStruct((_DIM, _BATCH), jnp.float32),
        mesh=mesh,
        scratch_types=[
            pltpu.VMEM((_VOCAB,), jnp.float32),
            pltpu.VMEM((_CTX, _BCHUNK), jnp.int32),
            pltpu.VMEM((_CTX, _BCHUNK), jnp.int32),
            pltpu.VMEM((_D_PER_W, _BATCH), jnp.float32),
            pltpu.SemaphoreType.DMA,
            pltpu.SemaphoreType.DMA,
            pltpu.SemaphoreType.DMA,
        ],
        compiler_params=pltpu.CompilerParams(
            use_tc_tiling_on_sc=True, needs_layout_passes=False,
        ),
    )(ctxT, tableT)


_VB = 4096


def _mm_kernel(wt_ref, x_ref, b_ref, o_ref):
    o_ref[...] = lax.dot_general(
        wt_ref[...], x_ref[...],
        dimension_numbers=(((0,), (0,)), ((), ())),
        preferred_element_type=jnp.float32,
    ) + b_ref[...].T


@jax.jit
def _project_t(WT, poolT, b2):
    return pl.pallas_call(
        _mm_kernel,
        grid=(pl.cdiv(_VOCAB, _VB),),
        in_specs=[
            pl.BlockSpec((_DIM, _VB), lambda i: (0, i)),
            pl.BlockSpec((_DIM, _BATCH), lambda i: (0, 0)),
            pl.BlockSpec((1, _VB), lambda i: (0, i)),
        ],
        out_specs=pl.BlockSpec((_VB, _BATCH), lambda i: (i, 0)),
        out_shape=jax.ShapeDtypeStruct((_VOCAB, _BATCH), jnp.float32),
    )(WT, poolT, b2)


def kernel(context, emb_table, W, b):
    context = context.astype(jnp.int32)
    poolT = _sc_pool_t(context.T, emb_table.T)
    outT = _project_t(W.T, poolT, b.reshape(1, _VOCAB))
    return outT.T


# speedup vs baseline: 1.0201x; 1.0201x over previous
import jax
import jax.numpy as jnp
from jax import lax
from jax.experimental import pallas as pl
from jax.experimental.pallas import tpu as pltpu
from jax.experimental.pallas import tpu_sc as plsc

_VOCAB = 100000
_DIM = 64
_BATCH = 1024
_CTX = 50

_NC = 2
_NS = 16
_NW = _NC * _NS
_LANES = 16

_D_PER_W = _DIM // _NW
_BCHUNK = 128
_NBCHUNK = _BATCH // _BCHUNK
_NGROUP = _BCHUNK // _LANES


def _sc_pool_kernel(
    ctxT_hbm, tableT_hbm, out_hbm, row_v, idx0_v, idx1_v, out_v, semr, sem0, sem1
):
    wid = lax.axis_index("s") * _NC + lax.axis_index("c")
    inv_ctx = jnp.float32(1.0 / _CTX)
    sems = (sem0, sem1)
    bufs = (idx0_v, idx1_v)

    def stage_idx(c):
        return pltpu.async_copy(
            ctxT_hbm.at[:, pl.ds(c * _BCHUNK, _BCHUNK)],
            bufs[c % 2],
            sems[c % 2],
        )

    for k in range(_D_PER_W):
        d = wid * _D_PER_W + k
        row_cp = pltpu.async_copy(tableT_hbm.at[d], row_v, semr)
        idx_cp = stage_idx(0)
        for c in range(_NBCHUNK):
            idx_cp.wait()
            if c + 1 < _NBCHUNK:
                idx_cp = stage_idx(c + 1)
            if c == 0:
                row_cp.wait()
            buf = c % 2

            def jbody(j, accs):
                return tuple(
                    accs[g]
                    + plsc.load_gather(
                        row_v, [bufs[buf][j, pl.ds(g * _LANES, _LANES)]]
                    )
                    for g in range(_NGROUP)
                )

            accs = tuple(
                jnp.zeros((_LANES,), jnp.float32) for _ in range(_NGROUP)
            )
            accs = lax.fori_loop(0, _CTX, jbody, accs)
            for g in range(_NGROUP):
                out_v[k, pl.ds(c * _BCHUNK + g * _LANES, _LANES)] = (
                    accs[g] * inv_ctx
                )

    pltpu.sync_copy(out_v, out_hbm.at[pl.ds(wid * _D_PER_W, _D_PER_W)])


@jax.jit
def _sc_pool_t(ctxT, tableT):
    mesh = plsc.VectorSubcoreMesh(
        core_axis_name="c", subcore_axis_name="s",
        num_cores=_NC, num_subcores=_NS,
    )
    return pl.kernel(
        _sc_pool_kernel,
        out_type=jax.ShapeDtypeStruct((_DIM, _BATCH), jnp.float32),
        mesh=mesh,
        scratch_types=[
            pltpu.VMEM((_VOCAB,), jnp.float32),
            pltpu.VMEM((_CTX, _BCHUNK), jnp.int32),
            pltpu.VMEM((_CTX, _BCHUNK), jnp.int32),
            pltpu.VMEM((_D_PER_W, _BATCH), jnp.float32),
            pltpu.SemaphoreType.DMA,
            pltpu.SemaphoreType.DMA,
            pltpu.SemaphoreType.DMA,
        ],
        compiler_params=pltpu.CompilerParams(
            use_tc_tiling_on_sc=True, needs_layout_passes=False,
        ),
    )(ctxT, tableT)


_VB = 4096


def _mm_kernel(wt_ref, x_ref, b_ref, o_ref):
    o_ref[...] = lax.dot_general(
        wt_ref[...], x_ref[...],
        dimension_numbers=(((0,), (0,)), ((), ())),
        preferred_element_type=jnp.float32,
    ) + b_ref[...].T


@jax.jit
def _project_t(WT, poolT, b2):
    return pl.pallas_call(
        _mm_kernel,
        grid=(pl.cdiv(_VOCAB, _VB),),
        in_specs=[
            pl.BlockSpec((_DIM, _VB), lambda i: (0, i)),
            pl.BlockSpec((_DIM, _BATCH), lambda i: (0, 0)),
            pl.BlockSpec((1, _VB), lambda i: (0, i)),
        ],
        out_specs=pl.BlockSpec((_VB, _BATCH), lambda i: (i, 0)),
        out_shape=jax.ShapeDtypeStruct((_VOCAB, _BATCH), jnp.float32),
    )(WT, poolT, b2)


def kernel(context, emb_table, W, b):
    context = context.astype(jnp.int32)
    poolT = _sc_pool_t(context.T, emb_table.T)
    outT = _project_t(W.T, poolT, b.reshape(1, _VOCAB))
    return outT.T
